# baseline (speedup 1.0000x reference)
import functools

import jax
import jax.numpy as jnp
from jax import lax
from jax.experimental import pallas as pl
from jax.experimental.pallas import tpu as pltpu
from jax.experimental.pallas import tpu_sc as plsc


def _build_kernel(N, D, NC, NCH, CH):
    n_per_w = NCH * CH

    mesh = plsc.VectorSubcoreMesh(core_axis_name="c", subcore_axis_name="s")

    @functools.partial(
        pl.kernel,
        mesh=mesh,
        out_type=jax.ShapeDtypeStruct((N, D), jnp.float32),
        scratch_types=[
            pltpu.VMEM((NCH, CH), jnp.int32),
            pltpu.VMEM((CH, D), jnp.float32),
            pltpu.SemaphoreType.DMA,
        ],
        compiler_params=pltpu.CompilerParams(use_tc_tiling_on_sc=False),
    )
    def k(idx_hbm, table_hbm, out_hbm, idx_v, buf, sem):
        wid = lax.axis_index("s") * NC + lax.axis_index("c")
        base = wid * n_per_w
        pltpu.sync_copy(idx_hbm.at[wid], idx_v)

        @pl.loop(0, NCH)
        def _chunk(c):
            pltpu.async_copy(table_hbm.at[idx_v.at[c]], buf, sem).wait()
            pltpu.sync_copy(buf, out_hbm.at[pl.ds(base + c * CH, CH)])

    return k


def kernel(x, table):
    B, H = x.shape
    V, D = table.shape
    N = B * H

    info = plsc.get_sparse_core_info()
    NC, NS = info.num_cores, info.num_subcores
    NW = NC * NS
    CH = 128
    NCH = N // (NW * CH)
    assert NW * NCH * CH == N

    idx = x.reshape(NW, NCH, CH)
    out = _build_kernel(N, D, NC, NCH, CH)(idx, table)
    return out.reshape(B, H, D)


# speedup vs baseline: 1.0457x; 1.0457x over previous
import functools

import jax
import jax.numpy as jnp
from jax import lax
from jax.experimental import pallas as pl
from jax.experimental.pallas import tpu as pltpu
from jax.experimental.pallas import tpu_sc as plsc

_NB = 10
_PF = 5


def _build_kernel(N, D, NC, NCH, CH):
    n_per_w = NCH * CH
    NB, PF = _NB, _PF
    assert NCH % NB == 0 and NCH >= 2 * NB

    mesh = plsc.VectorSubcoreMesh(core_axis_name="c", subcore_axis_name="s")

    @functools.partial(
        pl.kernel,
        mesh=mesh,
        out_type=jax.ShapeDtypeStruct((N, D), jnp.float32),
        scratch_types=[
            pltpu.VMEM((NCH, CH), jnp.int32),
            *[pltpu.VMEM((CH, D), jnp.float32) for _ in range(NB)],
            *[pltpu.SemaphoreType.DMA for _ in range(2 * NB)],
        ],
        compiler_params=pltpu.CompilerParams(use_tc_tiling_on_sc=False),
    )
    def k(idx_hbm, table_hbm, out_hbm, idx_v, *rest):
        bufs = rest[:NB]
        gsems = rest[NB:2 * NB]
        ssems = rest[2 * NB:]
        wid = lax.axis_index("s") * NC + lax.axis_index("c")
        base = wid * n_per_w
        pltpu.sync_copy(idx_hbm.at[wid], idx_v)

        def gather_start(c, s):
            pltpu.async_copy(table_hbm.at[idx_v.at[c]], bufs[s], gsems[s])

        def gather_wait(s):
            pltpu.make_async_copy(
                table_hbm.at[idx_v.at[0]], bufs[s], gsems[s]).wait()

        def scatter_start(c, s):
            pltpu.async_copy(
                bufs[s], out_hbm.at[pl.ds(base + c * CH, CH)], ssems[s])

        def scatter_wait(s):
            pltpu.make_async_copy(
                bufs[s], out_hbm.at[pl.ds(base, CH)], ssems[s]).wait()

        for c in range(PF):
            gather_start(c, c)
        for i in range(PF):
            gather_start(i + PF, i + PF)
            gather_wait(i)
            scatter_start(i, i)

        @pl.loop(PF, NCH - PF, step=NB)
        def _wave(w):
            for b in range(NB):
                i = w + b
                s = (PF + b) % NB
                sn = b
                scatter_wait(sn)
                gather_start(i + PF, sn)
                gather_wait(s)
                scatter_start(i, s)

        for i in range(NCH - PF, NCH):
            s = i % NB
            sn = (i + PF) % NB
            scatter_wait(sn)
            gather_wait(s)
            scatter_start(i, s)
        for i in range(NCH - PF, NCH):
            scatter_wait(i % NB)

    return k


def kernel(x, table):
    B, H = x.shape
    V, D = table.shape
    N = B * H

    info = plsc.get_sparse_core_info()
    NC, NS = info.num_cores, info.num_subcores
    NW = NC * NS
    CH = 128
    NCH = N // (NW * CH)
    assert NW * NCH * CH == N

    idx = x.reshape(NW, NCH, CH)
    out = _build_kernel(N, D, NC, NCH, CH)(idx, table)
    return out.reshape(B, H, D)
